# baseline (speedup 1.0000x reference)
import jax
import jax.numpy as jnp
from jax import lax
from jax.experimental import pallas as pl
from jax.experimental.pallas import tpu as pltpu
from jax.experimental.pallas import tpu_sc as plsc

B = 16384
S = 200
V = 100000
C = 16

NC = 2
NS = 16
NW = NC * NS
BPW = B // NW
GROUP = 16
TOK = GROUP * S
IDXW = 128
NGATH = TOK // IDXW
NGROUP = BPW // GROUP
NACC = 8

_LN2 = 0.6931471805599453


def _ln(v):
    bits = plsc.bitcast(v, jnp.int32)
    e = (bits >> 23) - 127
    m = plsc.bitcast((bits & 0x007FFFFF) | 0x3F800000, jnp.float32)
    z = (m - 1.0) / (m + 1.0)
    zz = z * z
    p = 1.0 + zz * (1.0 / 3.0 + zz * (1.0 / 5.0 + zz * (1.0 / 7.0)))
    return e.astype(jnp.float32) * _LN2 + 2.0 * z * p


def _sc_body(idx_hbm, emb_hbm, bias_hbm, out_hbm, idx_v, rows_v, acc_v,
             out_v, bias_v, gsem0, gsem1):
    wid = lax.axis_index("s") * NC + lax.axis_index("c")
    pltpu.sync_copy(bias_hbm, bias_v)
    bias_vec = bias_v[...]
    tile_tok0 = wid * (BPW * S)
    lanes = lax.iota(jnp.int32, 16)

    def stage_and_fire(buf, g, sem):
        pltpu.sync_copy(idx_hbm.at[pl.ds(tile_tok0 + g * TOK, TOK)],
                        idx_v.at[buf])
        for c in range(NGATH):
            pltpu.async_copy(
                emb_hbm.at[idx_v.at[buf, pl.ds(c * IDXW, IDXW)]],
                rows_v.at[buf, pl.ds(c * IDXW, IDXW)],
                sem,
            )

    def drain(buf, sem):
        pltpu.make_async_copy(emb_hbm.at[pl.ds(0, TOK)], rows_v.at[buf],
                              sem).wait()

    def accumulate(buf, g):
        def sample_body(i, _):
            base = i * S
            a = [jnp.zeros((16,), jnp.float32) for _ in range(NACC)]
            for j in range(S):
                a[j % NACC] = a[j % NACC] + rows_v[buf, base + j]
            a = [a[0] + a[1], a[2] + a[3], a[4] + a[5], a[6] + a[7]]
            acc_v[i] = ((a[0] + a[1]) + (a[2] + a[3])) + bias_vec
            return 0

        lax.fori_loop(0, GROUP, sample_body, 0)

        cols = [
            plsc.load_gather(acc_v, [lanes, jnp.full((16,), c, jnp.int32)])
            for c in range(C)
        ]
        m = cols[0]
        for c in range(1, C):
            m = jnp.maximum(m, cols[c])
        t = [cols[c] - m for c in range(C)]
        es = [jnp.exp(tc) for tc in t]
        ssum = es[0]
        for c in range(1, C):
            ssum = ssum + es[c]
        lse = _ln(ssum)
        for c in range(C):
            plsc.store_scatter(out_v, [lanes, jnp.full((16,), c, jnp.int32)],
                               t[c] - lse)
        pltpu.sync_copy(out_v,
                        out_hbm.at[pl.ds(wid * BPW + g * GROUP, GROUP)])

    stage_and_fire(0, 0, gsem0)

    def pair_body(gg, _):
        g0 = 2 * gg
        stage_and_fire(1, g0 + 1, gsem1)
        drain(0, gsem0)
        accumulate(0, g0)

        @pl.when(gg != NGROUP // 2 - 1)
        def _():
            stage_and_fire(0, g0 + 2, gsem0)

        drain(1, gsem1)
        accumulate(1, g0 + 1)
        return 0

    lax.fori_loop(0, NGROUP // 2, pair_body, 0)


_sc_lr = pl.kernel(
    _sc_body,
    out_type=jax.ShapeDtypeStruct((B, C), jnp.float32),
    mesh=plsc.VectorSubcoreMesh(
        core_axis_name="c", subcore_axis_name="s", num_cores=NC,
        num_subcores=NS),
    scratch_types=[
        pltpu.VMEM((2, TOK), jnp.int32),
        pltpu.VMEM((2, TOK, C), jnp.float32),
        pltpu.VMEM((GROUP, C), jnp.float32),
        pltpu.VMEM((GROUP, C), jnp.float32),
        pltpu.VMEM((C,), jnp.float32),
        pltpu.SemaphoreType.DMA,
        pltpu.SemaphoreType.DMA,
    ],
    compiler_params=pltpu.CompilerParams(use_tc_tiling_on_sc=False,
                                         needs_layout_passes=False),
)


def kernel(text, emb, bias):
    return _sc_lr(text.reshape(B * S), emb, bias)


# speedup vs baseline: 1.0046x; 1.0046x over previous
import jax
import jax.numpy as jnp
from jax import lax
from jax.experimental import pallas as pl
from jax.experimental.pallas import tpu as pltpu
from jax.experimental.pallas import tpu_sc as plsc

B = 16384
S = 200
V = 100000
C = 16

NC = 2
NS = 16
NW = NC * NS
BPW = B // NW
GROUP = 16
TOK = GROUP * S
IDXW = 128
NGATH = TOK // IDXW
NGROUP = BPW // GROUP
NACC = 8

_LN2 = 0.6931471805599453


def _ln(v):
    bits = plsc.bitcast(v, jnp.int32)
    e = (bits >> 23) - 127
    m = plsc.bitcast((bits & 0x007FFFFF) | 0x3F800000, jnp.float32)
    z = (m - 1.0) / (m + 1.0)
    zz = z * z
    p = 1.0 + zz * (1.0 / 3.0 + zz * (1.0 / 5.0 + zz * (1.0 / 7.0)))
    return e.astype(jnp.float32) * _LN2 + 2.0 * z * p


def _sc_body(text_hbm, emb_hbm, bias_hbm, out_hbm, idx_v, rows_v, acc_v,
             out_v, bias_v, gsem0, gsem1):
    wid = lax.axis_index("s") * NC + lax.axis_index("c")
    pltpu.sync_copy(bias_hbm, bias_v)
    bias_vec = bias_v[...]
    lanes = lax.iota(jnp.int32, 16)

    def stage_and_fire(buf, g, sem):
        pltpu.sync_copy(text_hbm.at[pl.ds(wid * BPW + g * GROUP, GROUP)],
                        idx_v.at[buf])
        for i in range(GROUP):
            pltpu.async_copy(
                emb_hbm.at[idx_v.at[buf, i, pl.ds(0, IDXW)]],
                rows_v.at[buf, pl.ds(i * S, IDXW)],
                sem,
            )
            pltpu.async_copy(
                emb_hbm.at[idx_v.at[buf, i, pl.ds(IDXW, S - IDXW)]],
                rows_v.at[buf, pl.ds(i * S + IDXW, S - IDXW)],
                sem,
            )

    def drain(buf, sem):
        pltpu.make_async_copy(emb_hbm.at[pl.ds(0, TOK)], rows_v.at[buf],
                              sem).wait()

    def accumulate(buf, g):
        def sample_body(i, _):
            base = i * S
            a = [jnp.zeros((16,), jnp.float32) for _ in range(NACC)]
            for j in range(S):
                a[j % NACC] = a[j % NACC] + rows_v[buf, base + j]
            a = [a[0] + a[1], a[2] + a[3], a[4] + a[5], a[6] + a[7]]
            acc_v[i] = ((a[0] + a[1]) + (a[2] + a[3])) + bias_vec
            return 0

        lax.fori_loop(0, GROUP, sample_body, 0)

        cols = [
            plsc.load_gather(acc_v, [lanes, jnp.full((16,), c, jnp.int32)])
            for c in range(C)
        ]
        m = cols[0]
        for c in range(1, C):
            m = jnp.maximum(m, cols[c])
        t = [cols[c] - m for c in range(C)]
        es = [jnp.exp(tc) for tc in t]
        ssum = es[0]
        for c in range(1, C):
            ssum = ssum + es[c]
        lse = _ln(ssum)
        for c in range(C):
            plsc.store_scatter(out_v, [lanes, jnp.full((16,), c, jnp.int32)],
                               t[c] - lse)
        pltpu.sync_copy(out_v,
                        out_hbm.at[pl.ds(wid * BPW + g * GROUP, GROUP)])

    stage_and_fire(0, 0, gsem0)

    def pair_body(gg, _):
        g0 = 2 * gg
        stage_and_fire(1, g0 + 1, gsem1)
        drain(0, gsem0)
        accumulate(0, g0)

        @pl.when(gg != NGROUP // 2 - 1)
        def _():
            stage_and_fire(0, g0 + 2, gsem0)

        drain(1, gsem1)
        accumulate(1, g0 + 1)
        return 0

    lax.fori_loop(0, NGROUP // 2, pair_body, 0)


_sc_lr = pl.kernel(
    _sc_body,
    out_type=jax.ShapeDtypeStruct((B, C), jnp.float32),
    mesh=plsc.VectorSubcoreMesh(
        core_axis_name="c", subcore_axis_name="s", num_cores=NC,
        num_subcores=NS),
    scratch_types=[
        pltpu.VMEM((2, GROUP, S), jnp.int32),
        pltpu.VMEM((2, TOK, C), jnp.float32),
        pltpu.VMEM((GROUP, C), jnp.float32),
        pltpu.VMEM((GROUP, C), jnp.float32),
        pltpu.VMEM((C,), jnp.float32),
        pltpu.SemaphoreType.DMA,
        pltpu.SemaphoreType.DMA,
    ],
    compiler_params=pltpu.CompilerParams(use_tc_tiling_on_sc=False,
                                         needs_layout_passes=False),
)


def kernel(text, emb, bias):
    return _sc_lr(text, emb, bias)


# speedup vs baseline: 1.0695x; 1.0646x over previous
import jax
import jax.numpy as jnp
from jax import lax
from jax.experimental import pallas as pl
from jax.experimental.pallas import tpu as pltpu
from jax.experimental.pallas import tpu_sc as plsc

B = 16384
S = 200
V = 100000
C = 16

NC = 2
NS = 16
NW = NC * NS
BPW = B // NW
GROUP = 16
TOK = GROUP * S
IDXW = 128
NGATH = TOK // IDXW
NROWS = B * S // IDXW
RPW = NROWS // NW
STG = NGATH + 7
NGROUP = BPW // GROUP
NACC = 8


def _sc_body(idx_hbm, emb_hbm, bias_hbm, out_hbm, idx_v, rows_v, acc_v,
             bias_v, gsem0, gsem1):
    wid = lax.axis_index("s") * NC + lax.axis_index("c")
    pltpu.sync_copy(bias_hbm, bias_v)
    bias_vec = bias_v[...]
    row0 = wid * RPW

    def stage_and_fire(buf, g, sem):
        lead = lax.rem(g, 8)
        base = row0 + g * NGATH - lead
        pltpu.sync_copy(idx_hbm.at[pl.ds(base, STG)], idx_v.at[buf])
        for c in range(NGATH):
            pltpu.async_copy(
                emb_hbm.at[idx_v.at[buf, lead + c]],
                rows_v.at[buf, pl.ds(c * IDXW, IDXW)],
                sem,
            )

    def drain(buf, sem):
        pltpu.make_async_copy(emb_hbm.at[pl.ds(0, TOK)], rows_v.at[buf],
                              sem).wait()

    def accumulate(buf, g):
        def sample_body(i, _):
            base = i * S
            a = [jnp.zeros((16,), jnp.float32) for _ in range(NACC)]
            for j in range(S):
                a[j % NACC] = a[j % NACC] + rows_v[buf, base + j]
            a = [a[0] + a[1], a[2] + a[3], a[4] + a[5], a[6] + a[7]]
            acc_v[i] = ((a[0] + a[1]) + (a[2] + a[3])) + bias_vec
            return 0

        lax.fori_loop(0, GROUP, sample_body, 0)
        pltpu.sync_copy(acc_v,
                        out_hbm.at[pl.ds(wid * BPW + g * GROUP, GROUP)])

    stage_and_fire(0, 0, gsem0)

    def pair_body(gg, _):
        g0 = 2 * gg
        stage_and_fire(1, g0 + 1, gsem1)
        drain(0, gsem0)
        accumulate(0, g0)

        @pl.when(gg != NGROUP // 2 - 1)
        def _():
            stage_and_fire(0, g0 + 2, gsem0)

        drain(1, gsem1)
        accumulate(1, g0 + 1)
        return 0

    lax.fori_loop(0, NGROUP // 2, pair_body, 0)


_sc_pool = pl.kernel(
    _sc_body,
    out_type=jax.ShapeDtypeStruct((B, C), jnp.float32),
    mesh=plsc.VectorSubcoreMesh(
        core_axis_name="c", subcore_axis_name="s", num_cores=NC,
        num_subcores=NS),
    scratch_types=[
        pltpu.VMEM((2, STG, IDXW), jnp.int32),
        pltpu.VMEM((2, TOK, C), jnp.float32),
        pltpu.VMEM((GROUP, C), jnp.float32),
        pltpu.VMEM((C,), jnp.float32),
        pltpu.SemaphoreType.DMA,
        pltpu.SemaphoreType.DMA,
    ],
    compiler_params=pltpu.CompilerParams(use_tc_tiling_on_sc=False,
                                         needs_layout_passes=False),
)


def _tc_body(x_ref, o_ref):
    x = x_ref[...]
    m = jnp.max(x, axis=-1, keepdims=True)
    e = jnp.exp(x - m)
    lse = jnp.log(jnp.sum(e, axis=-1, keepdims=True))
    o_ref[...] = (x - m) - lse


_TCBLK = 2048
_tc_logsoftmax = pl.pallas_call(
    _tc_body,
    out_shape=jax.ShapeDtypeStruct((B, C), jnp.float32),
    grid=(B // _TCBLK,),
    in_specs=[pl.BlockSpec((_TCBLK, C), lambda i: (i, 0))],
    out_specs=pl.BlockSpec((_TCBLK, C), lambda i: (i, 0)),
)


def kernel(text, emb, bias):
    logits = _sc_pool(text.reshape(NROWS, IDXW), emb, bias)
    return _tc_logsoftmax(logits)
